# baseline (speedup 1.0000x reference)
import functools

import jax
import jax.numpy as jnp
from jax import lax
from jax.experimental import pallas as pl
from jax.experimental.pallas import tpu as pltpu
from jax.experimental.pallas import tpu_sc as plsc

N = 10000
E = 320000
D_IN = 128
D_H = 128
D_OUT = 64

NC = 2
NS = 16
NW = NC * NS

CH = 128
NCHUNK = 80
EPW = NCHUNK * CH
E_PAD = EPW * NW
N_PAD = 10240
RPT = N_PAD // NS
DEGW = 16
NPAIR = NCHUNK // 2
NRING = 8

CNT0 = 120
CNT1 = 40


def _make_sc_agg(D, with_deg):
    mesh = plsc.VectorSubcoreMesh(core_axis_name="c", subcore_axis_name="s")
    out_type = [jax.ShapeDtypeStruct((NC, N_PAD, D), jnp.float32)]
    scratch = [
        pltpu.VMEM((NRING, CH), jnp.int32),
        pltpu.VMEM((NRING, CH), jnp.int32),
        pltpu.VMEM((CH, D), jnp.float32),
        pltpu.VMEM((CH, D), jnp.float32),
        pltpu.VMEM_SHARED((N_PAD, D), jnp.float32),
        pltpu.SemaphoreType.DMA,
        pltpu.SemaphoreType.DMA,
        pltpu.SemaphoreType.DMA,
    ]
    if with_deg:
        out_type.append(jax.ShapeDtypeStruct((NC, N_PAD, DEGW), jnp.float32))
        scratch += [
            pltpu.VMEM((64, DEGW), jnp.float32),
            pltpu.VMEM((CH, DEGW), jnp.float32),
            pltpu.VMEM_SHARED((N_PAD, DEGW), jnp.float32),
            pltpu.SemaphoreType.DMA,
            pltpu.SemaphoreType.DMA,
        ]

    def body(table_hbm, edge_hbm, *refs):
        if with_deg:
            (agg_out, deg_out, src_rg, dst_rg, rows0, rows1, acc_sh,
             isem, gsem0, gsem1, zdeg, ones_v, deg_sh, dsem0, dsem1) = refs
        else:
            (agg_out, src_rg, dst_rg, rows0, rows1, acc_sh,
             isem, gsem0, gsem1) = refs
        cid = lax.axis_index("c")
        sid = lax.axis_index("s")
        nch = lax.select(cid == 0, CNT0, CNT1)
        npair = nch // 2
        cstart = lax.select(cid == 0, sid * CNT0, NS * CNT0 + sid * CNT1)
        ebase = cstart * CH

        def idx_load(g):
            r = lax.rem(g, NRING)
            pltpu.async_copy(edge_hbm.at[0, pl.ds(ebase + g * CH, CH)],
                             src_rg.at[r], isem)
            pltpu.async_copy(edge_hbm.at[1, pl.ds(ebase + g * CH, CH)],
                             dst_rg.at[r], isem)

        def idx_wait():
            for _ in range(4):
                pltpu.make_async_copy(edge_hbm.at[0, pl.ds(0, CH)],
                                      src_rg.at[0], isem).wait()

        idx_load(0)
        idx_load(1)

        scope = jax.named_scope
        zero16 = jnp.zeros((16,), jnp.float32)
        _z = scope("zinit"); _z.__enter__()
        def zrow(r, carry):
            for k in range(D // 16):
                rows0[r, pl.ds(k * 16, 16)] = zero16
            return carry
        lax.fori_loop(0, CH, zrow, 0)

        def zcopy(j, carry):
            pltpu.sync_copy(rows0, acc_sh.at[pl.ds(sid * RPT + j * CH, CH)])
            return carry
        lax.fori_loop(0, RPT // CH, zcopy, 0)

        if with_deg:
            one16 = jnp.ones((16,), jnp.float32)

            def zdeg_row(r, carry):
                zdeg[r, :] = zero16
                return carry
            lax.fori_loop(0, 64, zdeg_row, 0)

            def ones_row(r, carry):
                ones_v[r, :] = one16
                return carry
            lax.fori_loop(0, CH, ones_row, 0)

            def zdcopy(j, carry):
                pltpu.sync_copy(zdeg,
                                deg_sh.at[pl.ds(sid * RPT + j * 64, 64)])
                return carry
            lax.fori_loop(0, RPT // 64, zdcopy, 0)

        _z.__exit__(None, None, None)
        idx_wait()
        _b = scope("initbar"); _b.__enter__()
        plsc.subcore_barrier()
        _b.__exit__(None, None, None)
        _m = scope("mainloop"); _m.__enter__()

        def gather(g, rows, sem):
            r = lax.rem(g, NRING)
            return pltpu.async_copy(table_hbm.at[src_rg.at[r]], rows, sem)

        def gwait(rows, sem):
            pltpu.make_async_copy(table_hbm.at[src_rg.at[0]], rows,
                                  sem).wait()

        def scat(g, rows):
            r = lax.rem(g, NRING)
            pltpu.sync_copy(rows, acc_sh.at[dst_rg.at[r]], add=True)

        if with_deg:
            def deg_scat(g, dsem):
                r = lax.rem(g, NRING)
                pltpu.async_copy(ones_v, deg_sh.at[dst_rg.at[r]], dsem,
                                 add=True)

            def deg_wait(dsem):
                pltpu.make_async_copy(ones_v, deg_sh.at[dst_rg.at[0]],
                                      dsem).wait()

        gather(0, rows0, gsem0)

        def pair(i, carry):
            a = 2 * i
            b = a + 1

            @pl.when(i < npair - 1)
            def _():
                idx_load(a + 2)
                idx_load(b + 2)

            gwait(rows0, gsem0)
            gather(b, rows1, gsem1)
            scat(a, rows0)
            if with_deg:

                @pl.when(i > 0)
                def _():
                    deg_wait(dsem0)
                deg_scat(a, dsem0)

            @pl.when(i < npair - 1)
            def _():
                idx_wait()

            gwait(rows1, gsem1)

            @pl.when(i < npair - 1)
            def _():
                gather(a + 2, rows0, gsem0)
            scat(b, rows1)
            if with_deg:

                @pl.when(i > 0)
                def _():
                    deg_wait(dsem1)
                deg_scat(b, dsem1)
            return carry
        lax.fori_loop(0, npair, pair, 0)
        if with_deg:
            deg_wait(dsem0)
            deg_wait(dsem1)

        _m.__exit__(None, None, None)
        _b2 = scope("endbar"); _b2.__enter__()
        plsc.subcore_barrier()
        _b2.__exit__(None, None, None)
        _w = scope("writeout"); _w.__enter__()
        def wpiece(j, stage, sem):
            off = sid * RPT + j * CH
            pltpu.sync_copy(acc_sh.at[pl.ds(off, CH)], stage)
            return pltpu.async_copy(stage, agg_out.at[cid, pl.ds(off, CH)],
                                    sem)
        w = wpiece(0, rows0, gsem0)
        for j in range(1, RPT // CH):
            stage, sem = (rows0, gsem0) if j % 2 == 0 else (rows1, gsem1)
            w2 = wpiece(j, stage, sem)
            w.wait()
            w = w2
        w.wait()
        if with_deg:
            for j in range(RPT // 64):
                off = sid * RPT + j * 64
                pltpu.sync_copy(deg_sh.at[pl.ds(off, 64)], zdeg)
                pltpu.sync_copy(zdeg, deg_out.at[cid, pl.ds(off, 64)])

        _w.__exit__(None, None, None)

    return pl.kernel(
        body, out_type=out_type, mesh=mesh, scratch_types=scratch,
        compiler_params=pltpu.CompilerParams(use_tc_tiling_on_sc=False))


_sc_agg1 = _make_sc_agg(D_IN, with_deg=True)
_sc_agg2 = _make_sc_agg(D_OUT, with_deg=False)


BN = 1000


def _tc1_body(x_ref, agg_ref, degp_ref, ws1_ref, wn1_ref, b1_ref,
              ws2_ref, wn2_ref, b2_ref, wl_ref, blin_ref,
              h1p_ref, h1s_ref):
    x = x_ref[...]
    agg = agg_ref[0] + agg_ref[1]
    dp = degp_ref[...]
    deg = dp[0, :, 0] + dp[1, :, 0]
    inv = 1.0 / jnp.maximum(deg, 1.0)
    hn = agg * inv[:, None]
    dn = (((1,), (1,)), ((), ()))
    h1 = lax.dot_general(x, ws1_ref[...], dn)
    h1 = h1 + lax.dot_general(hn, wn1_ref[...], dn)
    h1 = jnp.maximum(h1 + b1_ref[...], 0.0)
    wl = wl_ref[...]
    wc_n = jnp.dot(wl, wn2_ref[...])
    wc_s = jnp.dot(wl, ws2_ref[...])
    bc = lax.dot_general(b2_ref[...], wl, dn) + blin_ref[...]
    h1p_ref[...] = lax.dot_general(h1, wc_n, dn)
    h1s_ref[...] = lax.dot_general(h1, wc_s, dn) + bc


def _tc1(x, agg, degp, ws1, wn1, b1, ws2, wn2, b2, wl, blin):
    grid = (N // BN,)
    return pl.pallas_call(
        _tc1_body,
        grid=grid,
        in_specs=[
            pl.BlockSpec((BN, D_IN), lambda i: (i, 0)),
            pl.BlockSpec((NC, BN, D_IN), lambda i: (0, i, 0)),
            pl.BlockSpec((NC, BN, DEGW), lambda i: (0, i, 0)),
            pl.BlockSpec((D_H, D_IN), lambda i: (0, 0)),
            pl.BlockSpec((D_H, D_IN), lambda i: (0, 0)),
            pl.BlockSpec((1, D_H), lambda i: (0, 0)),
            pl.BlockSpec((D_OUT, D_H), lambda i: (0, 0)),
            pl.BlockSpec((D_OUT, D_H), lambda i: (0, 0)),
            pl.BlockSpec((1, D_OUT), lambda i: (0, 0)),
            pl.BlockSpec((D_OUT, D_OUT), lambda i: (0, 0)),
            pl.BlockSpec((1, D_OUT), lambda i: (0, 0)),
        ],
        out_specs=[
            pl.BlockSpec((BN, D_OUT), lambda i: (i, 0)),
            pl.BlockSpec((BN, D_OUT), lambda i: (i, 0)),
        ],
        out_shape=[
            jax.ShapeDtypeStruct((N, D_OUT), jnp.float32),
            jax.ShapeDtypeStruct((N, D_OUT), jnp.float32),
        ],
    )(x, agg, degp, ws1, wn1, b1, ws2, wn2, b2, wl, blin)


def _tc2_body(h1s_ref, agg2_ref, degp_ref, out_ref):
    dp = degp_ref[...]
    deg = dp[0, :, 0] + dp[1, :, 0]
    inv = 1.0 / jnp.maximum(deg, 1.0)
    q = agg2_ref[0] + agg2_ref[1]
    out_ref[...] = h1s_ref[...] + q * inv[:, None]


def _tc2(h1s, agg2, degp):
    grid = (N // BN,)
    return pl.pallas_call(
        _tc2_body,
        grid=grid,
        in_specs=[
            pl.BlockSpec((BN, D_OUT), lambda i: (i, 0)),
            pl.BlockSpec((NC, BN, D_OUT), lambda i: (0, i, 0)),
            pl.BlockSpec((NC, BN, DEGW), lambda i: (0, i, 0)),
        ],
        out_specs=pl.BlockSpec((BN, D_OUT), lambda i: (i, 0)),
        out_shape=jax.ShapeDtypeStruct((N, D_OUT), jnp.float32),
    )(h1s, agg2, degp)


def kernel(in_feat, edge_index, W_self1, W_neigh1, b1, W_self2, W_neigh2,
           b2, W_lin, b_lin):
    pad = E_PAD - E
    pad_edges = jnp.stack([
        jnp.zeros((pad,), jnp.int32),
        N + jnp.arange(pad, dtype=jnp.int32) % (N_PAD - N),
    ])
    epad = jnp.concatenate([edge_index, pad_edges], axis=1)

    agg1, degp = _sc_agg1(in_feat, epad)
    h1p, h1s = _tc1(in_feat, agg1, degp, W_self1, W_neigh1,
                    b1.reshape(1, -1), W_self2, W_neigh2,
                    b2.reshape(1, -1), W_lin, b_lin.reshape(1, -1))
    (agg2,) = _sc_agg2(h1p, epad)
    return _tc2(h1s, agg2, degp)


# speedup vs baseline: 2.5158x; 2.5158x over previous
import functools

import jax
import jax.numpy as jnp
from jax import lax
from jax.experimental import pallas as pl
from jax.experimental.pallas import tpu as pltpu
from jax.experimental.pallas import tpu_sc as plsc

N = 10000
E = 320000
D_IN = 128
D_H = 128
D_OUT = 64

NC = 2
NS = 16
NW = NC * NS

CH = 128
NCHUNK = 80
EPW = NCHUNK * CH
E_PAD = EPW * NW
N_PAD = 10240
RPT = N_PAD // NS
DEGW = 16
NPAIR = NCHUNK // 2
NRING = 8

CNT0 = 80
CNT1 = 80


def _make_sc_agg(D, with_deg):
    mesh = plsc.VectorSubcoreMesh(core_axis_name="c", subcore_axis_name="s")
    out_type = [jax.ShapeDtypeStruct((NC, N_PAD, D), jnp.float32)]
    scratch = [
        pltpu.VMEM((NRING, CH), jnp.int32),
        pltpu.VMEM((NRING, CH), jnp.int32),
        pltpu.VMEM((CH, D), jnp.float32),
        pltpu.VMEM((CH, D), jnp.float32),
        pltpu.VMEM_SHARED((N_PAD, D), jnp.float32),
        pltpu.SemaphoreType.DMA,
        pltpu.SemaphoreType.DMA,
        pltpu.SemaphoreType.DMA,
    ]
    if with_deg:
        out_type.append(jax.ShapeDtypeStruct((NC, N_PAD, DEGW), jnp.float32))
        scratch += [
            pltpu.VMEM((64, DEGW), jnp.float32),
            pltpu.VMEM((CH, DEGW), jnp.float32),
            pltpu.VMEM_SHARED((N_PAD, DEGW), jnp.float32),
            pltpu.SemaphoreType.DMA,
            pltpu.SemaphoreType.DMA,
        ]

    def body(table_hbm, edge_hbm, *refs):
        if with_deg:
            (agg_out, deg_out, src_rg, dst_rg, rows0, rows1, acc_sh,
             isem, gsem0, gsem1, zdeg, ones_v, deg_sh, dsem0, dsem1) = refs
        else:
            (agg_out, src_rg, dst_rg, rows0, rows1, acc_sh,
             isem, gsem0, gsem1) = refs
        cid = lax.axis_index("c")
        sid = lax.axis_index("s")
        nch = lax.select(cid == 0, CNT0, CNT1)
        npair = nch // 2
        cstart = lax.select(cid == 0, sid * CNT0, NS * CNT0 + sid * CNT1)
        ebase = cstart * CH

        def idx_load(g):
            r = lax.rem(g, NRING)
            pltpu.async_copy(edge_hbm.at[0, pl.ds(ebase + g * CH, CH)],
                             src_rg.at[r], isem)
            pltpu.async_copy(edge_hbm.at[1, pl.ds(ebase + g * CH, CH)],
                             dst_rg.at[r], isem)

        def idx_wait():
            for _ in range(4):
                pltpu.make_async_copy(edge_hbm.at[0, pl.ds(0, CH)],
                                      src_rg.at[0], isem).wait()

        idx_load(0)
        idx_load(1)

        scope = jax.named_scope
        zero16 = jnp.zeros((16,), jnp.float32)
        _z = scope("zinit"); _z.__enter__()
        def zrow(r, carry):
            for k in range(D // 16):
                rows0[r, pl.ds(k * 16, 16)] = zero16
            return carry
        lax.fori_loop(0, CH, zrow, 0)

        def zcopy(j, carry):
            pltpu.sync_copy(rows0, acc_sh.at[pl.ds(sid * RPT + j * CH, CH)])
            return carry
        lax.fori_loop(0, RPT // CH, zcopy, 0)

        if with_deg:
            one16 = jnp.ones((16,), jnp.float32)

            def zdeg_row(r, carry):
                zdeg[r, :] = zero16
                return carry
            lax.fori_loop(0, 64, zdeg_row, 0)

            def ones_row(r, carry):
                ones_v[r, :] = one16
                return carry
            lax.fori_loop(0, CH, ones_row, 0)

            def zdcopy(j, carry):
                pltpu.sync_copy(zdeg,
                                deg_sh.at[pl.ds(sid * RPT + j * 64, 64)])
                return carry
            lax.fori_loop(0, RPT // 64, zdcopy, 0)

        _z.__exit__(None, None, None)
        idx_wait()
        _b = scope("initbar"); _b.__enter__()
        plsc.subcore_barrier()
        _b.__exit__(None, None, None)
        _m = scope("mainloop"); _m.__enter__()

        def gather(g, rows, sem):
            r = lax.rem(g, NRING)
            return pltpu.async_copy(table_hbm.at[src_rg.at[r]], rows, sem)

        def gwait(rows, sem):
            pltpu.make_async_copy(table_hbm.at[src_rg.at[0]], rows,
                                  sem).wait()

        def scat(g, rows):
            r = lax.rem(g, NRING)
            pltpu.sync_copy(rows, acc_sh.at[dst_rg.at[r]], add=True)

        if with_deg:
            def deg_scat(g, dsem):
                r = lax.rem(g, NRING)
                pltpu.async_copy(ones_v, deg_sh.at[dst_rg.at[r]], dsem,
                                 add=True)

            def deg_wait(dsem):
                pltpu.make_async_copy(ones_v, deg_sh.at[dst_rg.at[0]],
                                      dsem).wait()

        gather(0, rows0, gsem0)

        def pair(i, carry):
            a = 2 * i
            b = a + 1

            @pl.when(i < npair - 1)
            def _():
                idx_load(a + 2)
                idx_load(b + 2)

            gwait(rows0, gsem0)
            gather(b, rows1, gsem1)
            scat(a, rows0)
            if with_deg:

                @pl.when(i > 0)
                def _():
                    deg_wait(dsem0)
                deg_scat(a, dsem0)

            @pl.when(i < npair - 1)
            def _():
                idx_wait()

            gwait(rows1, gsem1)

            @pl.when(i < npair - 1)
            def _():
                gather(a + 2, rows0, gsem0)
            scat(b, rows1)
            if with_deg:

                @pl.when(i > 0)
                def _():
                    deg_wait(dsem1)
                deg_scat(b, dsem1)
            return carry
        lax.fori_loop(0, npair, pair, 0)
        if with_deg:
            deg_wait(dsem0)
            deg_wait(dsem1)

        _m.__exit__(None, None, None)
        _b2 = scope("endbar"); _b2.__enter__()
        plsc.subcore_barrier()
        _b2.__exit__(None, None, None)
        _w = scope("writeout"); _w.__enter__()
        def wpiece(j, stage, sem):
            off = sid * RPT + j * CH
            pltpu.sync_copy(acc_sh.at[pl.ds(off, CH)], stage)
            return pltpu.async_copy(stage, agg_out.at[cid, pl.ds(off, CH)],
                                    sem)
        w = wpiece(0, rows0, gsem0)
        for j in range(1, RPT // CH):
            stage, sem = (rows0, gsem0) if j % 2 == 0 else (rows1, gsem1)
            w2 = wpiece(j, stage, sem)
            w.wait()
            w = w2
        w.wait()
        if with_deg:
            for j in range(RPT // 64):
                off = sid * RPT + j * 64
                pltpu.sync_copy(deg_sh.at[pl.ds(off, 64)], zdeg)
                pltpu.sync_copy(zdeg, deg_out.at[cid, pl.ds(off, 64)])

        _w.__exit__(None, None, None)

    return pl.kernel(
        body, out_type=out_type, mesh=mesh, scratch_types=scratch,
        compiler_params=pltpu.CompilerParams(use_tc_tiling_on_sc=False))


_sc_agg1 = _make_sc_agg(D_IN, with_deg=True)
_sc_agg2 = _make_sc_agg(D_OUT, with_deg=False)


BN = 1000


def _tc1_body(x_ref, agg_ref, degp_ref, ws1_ref, wn1_ref, b1_ref,
              ws2_ref, wn2_ref, b2_ref, wl_ref, blin_ref,
              h1p_ref, h1s_ref):
    x = x_ref[...]
    agg = agg_ref[0] + agg_ref[1]
    dp = degp_ref[...]
    deg = dp[0, :, 0] + dp[1, :, 0]
    inv = 1.0 / jnp.maximum(deg, 1.0)
    hn = agg * inv[:, None]
    dn = (((1,), (1,)), ((), ()))
    h1 = lax.dot_general(x, ws1_ref[...], dn)
    h1 = h1 + lax.dot_general(hn, wn1_ref[...], dn)
    h1 = jnp.maximum(h1 + b1_ref[...], 0.0)
    wl = wl_ref[...]
    wc_n = jnp.dot(wl, wn2_ref[...])
    wc_s = jnp.dot(wl, ws2_ref[...])
    bc = lax.dot_general(b2_ref[...], wl, dn) + blin_ref[...]
    h1p_ref[...] = lax.dot_general(h1, wc_n, dn)
    h1s_ref[...] = lax.dot_general(h1, wc_s, dn) + bc


def _tc1(x, agg, degp, ws1, wn1, b1, ws2, wn2, b2, wl, blin):
    grid = (N // BN,)
    return pl.pallas_call(
        _tc1_body,
        grid=grid,
        in_specs=[
            pl.BlockSpec((BN, D_IN), lambda i: (i, 0)),
            pl.BlockSpec((NC, BN, D_IN), lambda i: (0, i, 0)),
            pl.BlockSpec((NC, BN, DEGW), lambda i: (0, i, 0)),
            pl.BlockSpec((D_H, D_IN), lambda i: (0, 0)),
            pl.BlockSpec((D_H, D_IN), lambda i: (0, 0)),
            pl.BlockSpec((1, D_H), lambda i: (0, 0)),
            pl.BlockSpec((D_OUT, D_H), lambda i: (0, 0)),
            pl.BlockSpec((D_OUT, D_H), lambda i: (0, 0)),
            pl.BlockSpec((1, D_OUT), lambda i: (0, 0)),
            pl.BlockSpec((D_OUT, D_OUT), lambda i: (0, 0)),
            pl.BlockSpec((1, D_OUT), lambda i: (0, 0)),
        ],
        out_specs=[
            pl.BlockSpec((BN, D_OUT), lambda i: (i, 0)),
            pl.BlockSpec((BN, D_OUT), lambda i: (i, 0)),
        ],
        out_shape=[
            jax.ShapeDtypeStruct((N, D_OUT), jnp.float32),
            jax.ShapeDtypeStruct((N, D_OUT), jnp.float32),
        ],
    )(x, agg, degp, ws1, wn1, b1, ws2, wn2, b2, wl, blin)


def _tc2_body(h1s_ref, agg2_ref, degp_ref, out_ref):
    dp = degp_ref[...]
    deg = dp[0, :, 0] + dp[1, :, 0]
    inv = 1.0 / jnp.maximum(deg, 1.0)
    q = agg2_ref[0] + agg2_ref[1]
    out_ref[...] = h1s_ref[...] + q * inv[:, None]


def _tc2(h1s, agg2, degp):
    grid = (N // BN,)
    return pl.pallas_call(
        _tc2_body,
        grid=grid,
        in_specs=[
            pl.BlockSpec((BN, D_OUT), lambda i: (i, 0)),
            pl.BlockSpec((NC, BN, D_OUT), lambda i: (0, i, 0)),
            pl.BlockSpec((NC, BN, DEGW), lambda i: (0, i, 0)),
        ],
        out_specs=pl.BlockSpec((BN, D_OUT), lambda i: (i, 0)),
        out_shape=jax.ShapeDtypeStruct((N, D_OUT), jnp.float32),
    )(h1s, agg2, degp)


def kernel(in_feat, edge_index, W_self1, W_neigh1, b1, W_self2, W_neigh2,
           b2, W_lin, b_lin):
    pad = E_PAD - E
    ar = jnp.arange(pad, dtype=jnp.int32)
    pad_edges = jnp.stack([ar % N, N + ar % (N_PAD - N)])
    epad = jnp.concatenate([edge_index, pad_edges], axis=1)

    agg1, degp = _sc_agg1(in_feat, epad)
    h1p, h1s = _tc1(in_feat, agg1, degp, W_self1, W_neigh1,
                    b1.reshape(1, -1), W_self2, W_neigh2,
                    b2.reshape(1, -1), W_lin, b_lin.reshape(1, -1))
    (agg2,) = _sc_agg2(h1p, epad)
    return _tc2(h1s, agg2, degp)


# speedup vs baseline: 2.7395x; 1.0889x over previous
import functools

import jax
import jax.numpy as jnp
from jax import lax
from jax.experimental import pallas as pl
from jax.experimental.pallas import tpu as pltpu
from jax.experimental.pallas import tpu_sc as plsc

N = 10000
E = 320000
D_IN = 128
D_H = 128
D_OUT = 64

NC = 2
NS = 16
NW = NC * NS

CH = 128
NCHUNK = 80
EPW = NCHUNK * CH
E_PAD = EPW * NW
N_PAD = 10240
RPT = N_PAD // NS
DEGW = 16
NPAIR = NCHUNK // 2
NRING = 8

CNT0 = 80
CNT1 = 80


def _make_sc_agg(D, with_deg):
    mesh = plsc.VectorSubcoreMesh(core_axis_name="c", subcore_axis_name="s")
    out_type = [jax.ShapeDtypeStruct((NC, N_PAD, D), jnp.bfloat16)]
    scratch = [
        pltpu.VMEM((NRING, CH), jnp.int32),
        pltpu.VMEM((NRING, CH), jnp.int32),
        pltpu.VMEM((CH, D), jnp.bfloat16),
        pltpu.VMEM((CH, D), jnp.bfloat16),
        pltpu.VMEM_SHARED((N_PAD, D), jnp.bfloat16),
        pltpu.SemaphoreType.DMA,
        pltpu.SemaphoreType.DMA,
        pltpu.SemaphoreType.DMA,
    ]
    if with_deg:
        out_type.append(jax.ShapeDtypeStruct((NC, N_PAD, DEGW), jnp.float32))
        scratch += [
            pltpu.VMEM((64, DEGW), jnp.float32),
            pltpu.VMEM((CH, DEGW), jnp.float32),
            pltpu.VMEM_SHARED((N_PAD, DEGW), jnp.float32),
            pltpu.SemaphoreType.DMA,
            pltpu.SemaphoreType.DMA,
        ]

    def body(table_hbm, edge_hbm, *refs):
        if with_deg:
            (agg_out, deg_out, src_rg, dst_rg, rows0, rows1, acc_sh,
             isem, gsem0, gsem1, zdeg, ones_v, deg_sh, dsem0, dsem1) = refs
        else:
            (agg_out, src_rg, dst_rg, rows0, rows1, acc_sh,
             isem, gsem0, gsem1) = refs
        cid = lax.axis_index("c")
        sid = lax.axis_index("s")
        nch = lax.select(cid == 0, CNT0, CNT1)
        npair = nch // 2
        cstart = lax.select(cid == 0, sid * CNT0, NS * CNT0 + sid * CNT1)
        ebase = cstart * CH

        def idx_load(g):
            r = lax.rem(g, NRING)
            pltpu.async_copy(edge_hbm.at[0, pl.ds(ebase + g * CH, CH)],
                             src_rg.at[r], isem)
            pltpu.async_copy(edge_hbm.at[1, pl.ds(ebase + g * CH, CH)],
                             dst_rg.at[r], isem)

        def idx_wait():
            for _ in range(4):
                pltpu.make_async_copy(edge_hbm.at[0, pl.ds(0, CH)],
                                      src_rg.at[0], isem).wait()

        idx_load(0)
        idx_load(1)

        scope = jax.named_scope
        zero16 = jnp.zeros((16,), jnp.float32)
        zero32b = jnp.zeros((32,), jnp.bfloat16)
        _z = scope("zinit"); _z.__enter__()
        def zrow(r, carry):
            for k in range(D // 32):
                rows0[r, pl.ds(k * 32, 32)] = zero32b
            return carry
        lax.fori_loop(0, CH, zrow, 0)

        def zcopy(j, carry):
            pltpu.sync_copy(rows0, acc_sh.at[pl.ds(sid * RPT + j * CH, CH)])
            return carry
        lax.fori_loop(0, RPT // CH, zcopy, 0)

        if with_deg:
            one16 = jnp.ones((16,), jnp.float32)

            def zdeg_row(r, carry):
                zdeg[r, :] = zero16
                return carry
            lax.fori_loop(0, 64, zdeg_row, 0)

            def ones_row(r, carry):
                ones_v[r, :] = one16
                return carry
            lax.fori_loop(0, CH, ones_row, 0)

            def zdcopy(j, carry):
                pltpu.sync_copy(zdeg,
                                deg_sh.at[pl.ds(sid * RPT + j * 64, 64)])
                return carry
            lax.fori_loop(0, RPT // 64, zdcopy, 0)

        _z.__exit__(None, None, None)
        idx_wait()
        _b = scope("initbar"); _b.__enter__()
        plsc.subcore_barrier()
        _b.__exit__(None, None, None)
        _m = scope("mainloop"); _m.__enter__()

        def gather(g, rows, sem):
            r = lax.rem(g, NRING)
            return pltpu.async_copy(table_hbm.at[src_rg.at[r]], rows, sem)

        def gwait(rows, sem):
            pltpu.make_async_copy(table_hbm.at[src_rg.at[0]], rows,
                                  sem).wait()

        def scat(g, rows):
            r = lax.rem(g, NRING)
            pltpu.sync_copy(rows, acc_sh.at[dst_rg.at[r]], add=True)

        if with_deg:
            def deg_scat(g, dsem):
                r = lax.rem(g, NRING)
                pltpu.async_copy(ones_v, deg_sh.at[dst_rg.at[r]], dsem,
                                 add=True)

            def deg_wait(dsem):
                pltpu.make_async_copy(ones_v, deg_sh.at[dst_rg.at[0]],
                                      dsem).wait()

        gather(0, rows0, gsem0)

        def pair(i, carry):
            a = 2 * i
            b = a + 1

            @pl.when(i < npair - 1)
            def _():
                idx_load(a + 2)
                idx_load(b + 2)

            gwait(rows0, gsem0)
            gather(b, rows1, gsem1)
            scat(a, rows0)
            if with_deg:

                @pl.when(i > 0)
                def _():
                    deg_wait(dsem0)
                deg_scat(a, dsem0)

            @pl.when(i < npair - 1)
            def _():
                idx_wait()

            gwait(rows1, gsem1)

            @pl.when(i < npair - 1)
            def _():
                gather(a + 2, rows0, gsem0)
            scat(b, rows1)
            if with_deg:

                @pl.when(i > 0)
                def _():
                    deg_wait(dsem1)
                deg_scat(b, dsem1)
            return carry
        lax.fori_loop(0, npair, pair, 0)
        if with_deg:
            deg_wait(dsem0)
            deg_wait(dsem1)

        _m.__exit__(None, None, None)
        _b2 = scope("endbar"); _b2.__enter__()
        plsc.subcore_barrier()
        _b2.__exit__(None, None, None)
        _w = scope("writeout"); _w.__enter__()
        def wpiece(j, stage, sem):
            off = sid * RPT + j * CH
            pltpu.sync_copy(acc_sh.at[pl.ds(off, CH)], stage)
            return pltpu.async_copy(stage, agg_out.at[cid, pl.ds(off, CH)],
                                    sem)
        w = wpiece(0, rows0, gsem0)
        for j in range(1, RPT // CH):
            stage, sem = (rows0, gsem0) if j % 2 == 0 else (rows1, gsem1)
            w2 = wpiece(j, stage, sem)
            w.wait()
            w = w2
        w.wait()
        if with_deg:
            for j in range(RPT // 64):
                off = sid * RPT + j * 64
                pltpu.sync_copy(deg_sh.at[pl.ds(off, 64)], zdeg)
                pltpu.sync_copy(zdeg, deg_out.at[cid, pl.ds(off, 64)])

        _w.__exit__(None, None, None)

    return pl.kernel(
        body, out_type=out_type, mesh=mesh, scratch_types=scratch,
        compiler_params=pltpu.CompilerParams(use_tc_tiling_on_sc=False))


_sc_agg1 = _make_sc_agg(D_IN, with_deg=True)
_sc_agg2 = _make_sc_agg(D_OUT, with_deg=False)


BN = 1000


def _tc1_body(x_ref, agg_ref, degp_ref, ws1_ref, wn1_ref, b1_ref,
              ws2_ref, wn2_ref, b2_ref, wl_ref, blin_ref,
              h1p_ref, h1s_ref):
    x = x_ref[...]
    agg = (agg_ref[0].astype(jnp.float32) +
           agg_ref[1].astype(jnp.float32))
    dp = degp_ref[...]
    deg = dp[0, :, 0] + dp[1, :, 0]
    inv = 1.0 / jnp.maximum(deg, 1.0)
    hn = agg * inv[:, None]
    dn = (((1,), (1,)), ((), ()))
    h1 = lax.dot_general(x, ws1_ref[...], dn)
    h1 = h1 + lax.dot_general(hn, wn1_ref[...], dn)
    h1 = jnp.maximum(h1 + b1_ref[...], 0.0)
    wl = wl_ref[...]
    wc_n = jnp.dot(wl, wn2_ref[...])
    wc_s = jnp.dot(wl, ws2_ref[...])
    bc = lax.dot_general(b2_ref[...], wl, dn) + blin_ref[...]
    h1p_ref[...] = lax.dot_general(h1, wc_n, dn).astype(jnp.bfloat16)
    h1s_ref[...] = lax.dot_general(h1, wc_s, dn) + bc


def _tc1(x, agg, degp, ws1, wn1, b1, ws2, wn2, b2, wl, blin):
    grid = (N // BN,)
    return pl.pallas_call(
        _tc1_body,
        grid=grid,
        in_specs=[
            pl.BlockSpec((BN, D_IN), lambda i: (i, 0)),
            pl.BlockSpec((NC, BN, D_IN), lambda i: (0, i, 0)),
            pl.BlockSpec((NC, BN, DEGW), lambda i: (0, i, 0)),
            pl.BlockSpec((D_H, D_IN), lambda i: (0, 0)),
            pl.BlockSpec((D_H, D_IN), lambda i: (0, 0)),
            pl.BlockSpec((1, D_H), lambda i: (0, 0)),
            pl.BlockSpec((D_OUT, D_H), lambda i: (0, 0)),
            pl.BlockSpec((D_OUT, D_H), lambda i: (0, 0)),
            pl.BlockSpec((1, D_OUT), lambda i: (0, 0)),
            pl.BlockSpec((D_OUT, D_OUT), lambda i: (0, 0)),
            pl.BlockSpec((1, D_OUT), lambda i: (0, 0)),
        ],
        out_specs=[
            pl.BlockSpec((BN, D_OUT), lambda i: (i, 0)),
            pl.BlockSpec((BN, D_OUT), lambda i: (i, 0)),
        ],
        out_shape=[
            jax.ShapeDtypeStruct((N, D_OUT), jnp.bfloat16),
            jax.ShapeDtypeStruct((N, D_OUT), jnp.float32),
        ],
    )(x, agg, degp, ws1, wn1, b1, ws2, wn2, b2, wl, blin)


def _tc2_body(h1s_ref, agg2_ref, degp_ref, out_ref):
    dp = degp_ref[...]
    deg = dp[0, :, 0] + dp[1, :, 0]
    inv = 1.0 / jnp.maximum(deg, 1.0)
    q = agg2_ref[0].astype(jnp.float32) + agg2_ref[1].astype(jnp.float32)
    out_ref[...] = h1s_ref[...] + q * inv[:, None]


def _tc2(h1s, agg2, degp):
    grid = (N // BN,)
    return pl.pallas_call(
        _tc2_body,
        grid=grid,
        in_specs=[
            pl.BlockSpec((BN, D_OUT), lambda i: (i, 0)),
            pl.BlockSpec((NC, BN, D_OUT), lambda i: (0, i, 0)),
            pl.BlockSpec((NC, BN, DEGW), lambda i: (0, i, 0)),
        ],
        out_specs=pl.BlockSpec((BN, D_OUT), lambda i: (i, 0)),
        out_shape=jax.ShapeDtypeStruct((N, D_OUT), jnp.float32),
    )(h1s, agg2, degp)


def kernel(in_feat, edge_index, W_self1, W_neigh1, b1, W_self2, W_neigh2,
           b2, W_lin, b_lin):
    pad = E_PAD - E
    ar = jnp.arange(pad, dtype=jnp.int32)
    pad_edges = jnp.stack([ar % N, N + ar % (N_PAD - N)])
    epad = jnp.concatenate([edge_index, pad_edges], axis=1)

    agg1, degp = _sc_agg1(in_feat.astype(jnp.bfloat16), epad)
    h1p, h1s = _tc1(in_feat, agg1, degp, W_self1, W_neigh1,
                    b1.reshape(1, -1), W_self2, W_neigh2,
                    b2.reshape(1, -1), W_lin, b_lin.reshape(1, -1))
    (agg2,) = _sc_agg2(h1p, epad)
    return _tc2(h1s, agg2, degp)
